# Initial kernel scaffold; baseline (speedup 1.0000x reference)
import functools

import jax
import jax.numpy as jnp
from jax import lax
from jax.experimental import pallas as pl
from jax.experimental.pallas import tpu as pltpu
from jax.experimental.pallas import tpu_sc as plsc

N_NODES = 10000
FEAT = 128
EMB = 16
NCLS = 16
N_RELS = 8
N_BASES = 4
NS = 8192
E = 262144

NC = 2
NSUB = 16
NW = NC * NSUB
L = 16

TW = N_RELS * EMB + EMB
EPW = E // NW
ROWS_PW = EPW // 128
GRP = 8
NGRP = ROWS_PW // GRP

_mesh = plsc.VectorSubcoreMesh(
    core_axis_name="c", subcore_axis_name="s", num_cores=NC, num_subcores=NSUB
)


def _zero_f32(ref, nrows):
    z = jnp.zeros((L,), jnp.float32)

    def body(i, _):
        ref[i, :] = z
        return 0

    lax.fori_loop(0, nrows, body, 0)


def _zero_flat(ref, n):
    z = jnp.zeros((L,), jnp.float32)

    def body(i, _):
        ref[pl.ds(i * L, L)] = z
        return 0

    lax.fori_loop(0, n // L, body, 0)


@functools.partial(
    pl.kernel,
    out_type=[
        jax.ShapeDtypeStruct((NS, FEAT), jnp.float32),
        jax.ShapeDtypeStruct((NS,), jnp.float32),
        jax.ShapeDtypeStruct((NS,), jnp.float32),
    ],
    mesh=_mesh,
    scratch_types=[
        pltpu.VMEM((2, 128), jnp.int32),
        pltpu.VMEM((128, FEAT), jnp.float32),
        pltpu.VMEM((128, 128), jnp.int32),
        pltpu.VMEM((NS,), jnp.float32),
        pltpu.VMEM((NSUB, NS // NSUB), jnp.float32),
        pltpu.VMEM((NS // NSUB,), jnp.float32),
        pltpu.VMEM_SHARED((NSUB, NS), jnp.float32),
        pltpu.SemaphoreType.DMA,
    ],
)
def _sc_prep(embed_hbm, an_hbm, dst1_hbm, dst2_hbm,
             h0_hbm, norm1_hbm, norm2_hbm,
             an_v, hrows, dst_v, deg_v, tb_v, acc_v, deg_sh, sem):
    cid = lax.axis_index("c")
    sid = lax.axis_index("s")
    w = cid * NSUB + sid

    pltpu.sync_copy(an_hbm.at[pl.ds(w * 2, 2)], an_v)
    for j in range(2):
        pltpu.async_copy(embed_hbm.at[an_v.at[j]], hrows, sem).wait()
        pltpu.sync_copy(hrows, h0_hbm.at[pl.ds(w * 256 + j * 128, 128)])

    _zero_flat(deg_v, NS)
    ones = jnp.full((L,), 1.0, jnp.float32)

    def count(dst_hbm):
        pltpu.sync_copy(dst_hbm.at[pl.ds(sid * 128, 128)], dst_v)

        def body(i, _):
            for k in range(8):
                d16 = dst_v[i, pl.ds(k * L, L)]
                plsc.addupdate_scatter(deg_v, [d16], ones)
            return 0

        lax.fori_loop(0, 128, body, 0)

    @pl.when(cid == 0)
    def _():
        count(dst1_hbm)

    @pl.when(cid == 1)
    def _():
        count(dst2_hbm)

    pltpu.sync_copy(deg_v, deg_sh.at[sid])
    plsc.subcore_barrier()
    chunk = NS // NSUB
    for t in range(NSUB):
        pltpu.async_copy(deg_sh.at[t, pl.ds(sid * chunk, chunk)], tb_v.at[t], sem)
    for t in range(NSUB):
        pltpu.make_async_copy(
            deg_sh.at[t, pl.ds(sid * chunk, chunk)], tb_v.at[t], sem
        ).wait()

    def red(i, _):
        s = tb_v[0, pl.ds(i * L, L)]
        for t in range(1, NSUB):
            s = s + tb_v[t, pl.ds(i * L, L)]
        acc_v[pl.ds(i * L, L)] = 1.0 / jnp.maximum(s, 1.0)
        return 0

    lax.fori_loop(0, chunk // L, red, 0)

    @pl.when(cid == 0)
    def _():
        pltpu.sync_copy(acc_v, norm1_hbm.at[pl.ds(sid * chunk, chunk)])

    @pl.when(cid == 1)
    def _():
        pltpu.sync_copy(acc_v, norm2_hbm.at[pl.ds(sid * chunk, chunk)])


@functools.partial(
    pl.kernel,
    out_type=[
        jax.ShapeDtypeStruct((NS, EMB), jnp.float32),
        jax.ShapeDtypeStruct((NS, EMB), jnp.float32),
    ],
    mesh=_mesh,
    scratch_types=[
        pltpu.VMEM((ROWS_PW, 128), jnp.int32),
        pltpu.VMEM((ROWS_PW, 128), jnp.int32),
        pltpu.VMEM((ROWS_PW, 128), jnp.int32),
        pltpu.VMEM((GRP, 128, EMB), jnp.float32),
        pltpu.VMEM((NS // NSUB, EMB), jnp.float32),
        pltpu.VMEM_SHARED((NS, EMB), jnp.float32),
        pltpu.SemaphoreType.DMA,
    ],
)
def _sc_msg(table_hbm, src_hbm, rel_hbm, dst_hbm,
            agg0_hbm, agg1_hbm,
            src_v, rel_v, dst_v, rows, zbuf, agg_sh, sem):
    cid = lax.axis_index("c")
    sid = lax.axis_index("s")
    w = cid * NSUB + sid
    chunk = NS // NSUB

    _zero_f32(zbuf, chunk)
    pltpu.sync_copy(zbuf, agg_sh.at[pl.ds(sid * chunk, chunk)])
    plsc.subcore_barrier()

    pltpu.sync_copy(src_hbm.at[pl.ds(w * ROWS_PW, ROWS_PW)], src_v)
    pltpu.sync_copy(rel_hbm.at[pl.ds(w * ROWS_PW, ROWS_PW)], rel_v)
    pltpu.sync_copy(dst_hbm.at[pl.ds(w * ROWS_PW, ROWS_PW)], dst_v)

    nine = jnp.full((L,), N_RELS + 1, jnp.int32)

    def gidx(i, _):
        for k in range(128 // L):
            sl = pl.ds(k * L, L)
            src_v[i, sl] = src_v[i, sl] * nine + rel_v[i, sl]
        return 0

    lax.fori_loop(0, ROWS_PW, gidx, 0)

    def group(g, _):
        base = g * GRP
        for k in range(GRP):
            pltpu.async_copy(table_hbm.at[src_v.at[base + k]], rows.at[k], sem)
        for k in range(GRP):
            pltpu.make_async_copy(
                table_hbm.at[src_v.at[base + k]], rows.at[k], sem
            ).wait()
        for k in range(GRP):
            pltpu.sync_copy(rows.at[k], agg_sh.at[dst_v.at[base + k]], add=True)
        return 0

    lax.fori_loop(0, NGRP, group, 0)
    plsc.subcore_barrier()

    pltpu.sync_copy(agg_sh.at[pl.ds(sid * chunk, chunk)], zbuf)

    @pl.when(cid == 0)
    def _():
        pltpu.sync_copy(zbuf, agg0_hbm.at[pl.ds(sid * chunk, chunk)])

    @pl.when(cid == 1)
    def _():
        pltpu.sync_copy(zbuf, agg1_hbm.at[pl.ds(sid * chunk, chunk)])


def _wcat(basis_ref, comp_ref, root_ref):
    parts = []
    for r in range(N_RELS):
        wr = comp_ref[r, 0] * basis_ref[0]
        for b in range(1, N_BASES):
            wr = wr + comp_ref[r, b] * basis_ref[b]
        parts.append(wr)
    parts.append(root_ref[...])
    return jnp.concatenate(parts, axis=1)


def _tc_table1_body(h_ref, basis_ref, comp_ref, root_ref, out_ref):
    wc = _wcat(basis_ref, comp_ref, root_ref)
    out_ref[...] = jnp.dot(h_ref[...], wc, preferred_element_type=jnp.float32)


def _tc_combine1_body(a0_ref, a1_ref, norm_ref, t1_ref,
                      basis_ref, comp_ref, root_ref, out_ref):
    r1 = t1_ref[:, N_RELS * EMB:]
    h1 = jax.nn.relu((a0_ref[...] + a1_ref[...]) * norm_ref[...] + r1)
    wc = _wcat(basis_ref, comp_ref, root_ref)
    out_ref[...] = jnp.dot(h1, wc, preferred_element_type=jnp.float32)


def _tc_final_body(a0_ref, a1_ref, norm_ref, t2_ref, out_ref):
    r2 = t2_ref[:, N_RELS * EMB:]
    out_ref[...] = (a0_ref[...] + a1_ref[...]) * norm_ref[...] + r2


_smem_spec = pl.BlockSpec(memory_space=pltpu.SMEM)

_tc_table1 = pl.pallas_call(
    _tc_table1_body,
    out_shape=jax.ShapeDtypeStruct((NS, TW), jnp.float32),
    in_specs=[pl.BlockSpec(), pl.BlockSpec(), _smem_spec, pl.BlockSpec()],
)

_tc_combine1 = pl.pallas_call(
    _tc_combine1_body,
    out_shape=jax.ShapeDtypeStruct((NS, TW), jnp.float32),
    in_specs=[pl.BlockSpec(), pl.BlockSpec(), pl.BlockSpec(),
              pl.BlockSpec(), pl.BlockSpec(), _smem_spec, pl.BlockSpec()],
)

_tc_final = pl.pallas_call(
    _tc_final_body,
    out_shape=jax.ShapeDtypeStruct((NS, NCLS), jnp.float32),
)


def kernel(embed_X, after_nodes, edge_src1, edge_dst1, edge_rel1,
           edge_src2, edge_dst2, edge_rel2,
           basis1, comp1, root1, basis2, comp2, root2):
    i32 = jnp.int32
    an2 = after_nodes.astype(i32).reshape(NS // 128, 128)
    s1 = edge_src1.astype(i32).reshape(E // 128, 128)
    d1 = edge_dst1.astype(i32).reshape(E // 128, 128)
    r1 = edge_rel1.astype(i32).reshape(E // 128, 128)
    s2 = edge_src2.astype(i32).reshape(E // 128, 128)
    d2 = edge_dst2.astype(i32).reshape(E // 128, 128)
    r2 = edge_rel2.astype(i32).reshape(E // 128, 128)

    h0, norm1, norm2 = _sc_prep(embed_X, an2, d1, d2)
    norm1 = norm1.reshape(NS, 1)
    norm2 = norm2.reshape(NS, 1)

    table1 = _tc_table1(h0, basis1, comp1, root1)
    a0, a1 = _sc_msg(table1.reshape(NS * (N_RELS + 1), EMB), s1, r1, d1)
    table2 = _tc_combine1(a0, a1, norm1, table1, basis2, comp2, root2)
    b0, b1 = _sc_msg(table2.reshape(NS * (N_RELS + 1), EMB), s2, r2, d2)
    return _tc_final(b0, b1, norm2, table2)


# speedup vs baseline: 62.7986x; 62.7986x over previous
import functools

import jax
import jax.numpy as jnp
from jax import lax
from jax.experimental import pallas as pl
from jax.experimental.pallas import tpu as pltpu
from jax.experimental.pallas import tpu_sc as plsc

N_NODES = 10000
FEAT = 128
EMB = 16
NCLS = 16
N_RELS = 8
N_BASES = 4
NS = 8192
E = 262144

NC = 2
NSUB = 16
NW = NC * NSUB
L = 16

TW = N_RELS * EMB + EMB
EPW = E // NW
ROWS_PW = EPW // 128
GRP = 8
NGRP = ROWS_PW // GRP

_mesh = plsc.VectorSubcoreMesh(
    core_axis_name="c", subcore_axis_name="s", num_cores=NC, num_subcores=NSUB
)
_sc_params = pltpu.CompilerParams(
    needs_layout_passes=False, use_tc_tiling_on_sc=False
)


def _zero_f32(ref, nrows):
    z = jnp.zeros((L,), jnp.float32)

    def body(i, _):
        ref[i, :] = z
        return 0

    lax.fori_loop(0, nrows, body, 0)


def _zero_flat(ref, n):
    z = jnp.zeros((L,), jnp.float32)

    def body(i, _):
        ref[pl.ds(i * L, L)] = z
        return 0

    lax.fori_loop(0, n // L, body, 0)


@functools.partial(
    pl.kernel,
    out_type=[
        jax.ShapeDtypeStruct((NS, FEAT), jnp.float32),
        jax.ShapeDtypeStruct((NS,), jnp.float32),
        jax.ShapeDtypeStruct((NS,), jnp.float32),
    ],
    mesh=_mesh,
    scratch_types=[
        pltpu.VMEM((2, 128), jnp.int32),
        pltpu.VMEM((128, FEAT), jnp.float32),
        pltpu.VMEM((128, 128), jnp.int32),
        pltpu.VMEM((NS,), jnp.float32),
        pltpu.VMEM((NSUB, NS // NSUB), jnp.float32),
        pltpu.VMEM((NS // NSUB,), jnp.float32),
        pltpu.VMEM_SHARED((NSUB, NS), jnp.float32),
        pltpu.SemaphoreType.DMA,
    ],
    compiler_params=_sc_params,
)
def _sc_prep(embed_hbm, an_hbm, dst1_hbm, dst2_hbm,
             h0_hbm, norm1_hbm, norm2_hbm,
             an_v, hrows, dst_v, deg_v, tb_v, acc_v, deg_sh, sem):
    cid = lax.axis_index("c")
    sid = lax.axis_index("s")
    w = cid * NSUB + sid

    pltpu.sync_copy(an_hbm.at[pl.ds(w * 2, 2)], an_v)
    for j in range(2):
        pltpu.async_copy(embed_hbm.at[an_v.at[j]], hrows, sem).wait()
        pltpu.sync_copy(hrows, h0_hbm.at[pl.ds(w * 256 + j * 128, 128)])

    _zero_flat(deg_v, NS)
    ones = jnp.full((L,), 1.0, jnp.float32)

    def count(dst_hbm):
        pltpu.sync_copy(dst_hbm.at[pl.ds(sid * 128, 128)], dst_v)

        def body(i, _):
            for k in range(8):
                d16 = dst_v[i, pl.ds(k * L, L)]
                plsc.addupdate_scatter(deg_v, [d16], ones)
            return 0

        lax.fori_loop(0, 128, body, 0)

    @pl.when(cid == 0)
    def _():
        count(dst1_hbm)

    @pl.when(cid == 1)
    def _():
        count(dst2_hbm)

    pltpu.sync_copy(deg_v, deg_sh.at[sid])
    plsc.subcore_barrier()
    chunk = NS // NSUB
    for t in range(NSUB):
        pltpu.async_copy(deg_sh.at[t, pl.ds(sid * chunk, chunk)], tb_v.at[t], sem)
    for t in range(NSUB):
        pltpu.make_async_copy(
            deg_sh.at[t, pl.ds(sid * chunk, chunk)], tb_v.at[t], sem
        ).wait()

    def red(i, _):
        s = tb_v[0, pl.ds(i * L, L)]
        for t in range(1, NSUB):
            s = s + tb_v[t, pl.ds(i * L, L)]
        acc_v[pl.ds(i * L, L)] = 1.0 / jnp.maximum(s, 1.0)
        return 0

    lax.fori_loop(0, chunk // L, red, 0)

    @pl.when(cid == 0)
    def _():
        pltpu.sync_copy(acc_v, norm1_hbm.at[pl.ds(sid * chunk, chunk)])

    @pl.when(cid == 1)
    def _():
        pltpu.sync_copy(acc_v, norm2_hbm.at[pl.ds(sid * chunk, chunk)])


@functools.partial(
    pl.kernel,
    out_type=[
        jax.ShapeDtypeStruct((NS, EMB), jnp.float32),
        jax.ShapeDtypeStruct((NS, EMB), jnp.float32),
    ],
    mesh=_mesh,
    scratch_types=[
        pltpu.VMEM((ROWS_PW, 128), jnp.int32),
        pltpu.VMEM((ROWS_PW, 128), jnp.int32),
        pltpu.VMEM((ROWS_PW, 128), jnp.int32),
        pltpu.VMEM((GRP, 128, EMB), jnp.float32),
        pltpu.VMEM((NS // NSUB, EMB), jnp.float32),
        pltpu.VMEM_SHARED((NS, EMB), jnp.float32),
        pltpu.SemaphoreType.DMA,
    ],
    compiler_params=_sc_params,
)
def _sc_msg(table_hbm, src_hbm, rel_hbm, dst_hbm,
            agg0_hbm, agg1_hbm,
            src_v, rel_v, dst_v, rows, zbuf, agg_sh, sem):
    cid = lax.axis_index("c")
    sid = lax.axis_index("s")
    w = cid * NSUB + sid
    chunk = NS // NSUB

    _zero_f32(zbuf, chunk)
    pltpu.sync_copy(zbuf, agg_sh.at[pl.ds(sid * chunk, chunk)])
    plsc.subcore_barrier()

    pltpu.sync_copy(src_hbm.at[pl.ds(w * ROWS_PW, ROWS_PW)], src_v)
    pltpu.sync_copy(rel_hbm.at[pl.ds(w * ROWS_PW, ROWS_PW)], rel_v)
    pltpu.sync_copy(dst_hbm.at[pl.ds(w * ROWS_PW, ROWS_PW)], dst_v)

    nine = jnp.full((L,), N_RELS + 1, jnp.int32)

    def gidx(i, _):
        for k in range(128 // L):
            sl = pl.ds(k * L, L)
            src_v[i, sl] = src_v[i, sl] * nine + rel_v[i, sl]
        return 0

    lax.fori_loop(0, ROWS_PW, gidx, 0)

    def group(g, _):
        base = g * GRP
        for k in range(GRP):
            pltpu.async_copy(table_hbm.at[src_v.at[base + k]], rows.at[k], sem)
        for k in range(GRP):
            pltpu.make_async_copy(
                table_hbm.at[src_v.at[base + k]], rows.at[k], sem
            ).wait()
        for k in range(GRP):
            pltpu.sync_copy(rows.at[k], agg_sh.at[dst_v.at[base + k]], add=True)
        return 0

    lax.fori_loop(0, NGRP, group, 0)
    plsc.subcore_barrier()

    pltpu.sync_copy(agg_sh.at[pl.ds(sid * chunk, chunk)], zbuf)

    @pl.when(cid == 0)
    def _():
        pltpu.sync_copy(zbuf, agg0_hbm.at[pl.ds(sid * chunk, chunk)])

    @pl.when(cid == 1)
    def _():
        pltpu.sync_copy(zbuf, agg1_hbm.at[pl.ds(sid * chunk, chunk)])


def _wcat(basis_ref, comp_ref, root_ref):
    parts = []
    for r in range(N_RELS):
        wr = comp_ref[r, 0] * basis_ref[0]
        for b in range(1, N_BASES):
            wr = wr + comp_ref[r, b] * basis_ref[b]
        parts.append(wr)
    parts.append(root_ref[...])
    return jnp.concatenate(parts, axis=1)


def _tc_table1_body(h_ref, basis_ref, comp_ref, root_ref, out_ref):
    wc = _wcat(basis_ref, comp_ref, root_ref)
    out_ref[...] = jnp.dot(h_ref[...], wc, preferred_element_type=jnp.float32)


def _tc_combine1_body(a0_ref, a1_ref, norm_ref, t1_ref,
                      basis_ref, comp_ref, root_ref, out_ref):
    r1 = t1_ref[:, N_RELS * EMB:]
    h1 = jax.nn.relu((a0_ref[...] + a1_ref[...]) * norm_ref[...] + r1)
    wc = _wcat(basis_ref, comp_ref, root_ref)
    out_ref[...] = jnp.dot(h1, wc, preferred_element_type=jnp.float32)


def _tc_final_body(a0_ref, a1_ref, norm_ref, t2_ref, out_ref):
    r2 = t2_ref[:, N_RELS * EMB:]
    out_ref[...] = (a0_ref[...] + a1_ref[...]) * norm_ref[...] + r2


_smem_spec = pl.BlockSpec(memory_space=pltpu.SMEM)

_tc_table1 = pl.pallas_call(
    _tc_table1_body,
    out_shape=jax.ShapeDtypeStruct((NS, TW), jnp.float32),
    in_specs=[pl.BlockSpec(), pl.BlockSpec(), _smem_spec, pl.BlockSpec()],
)

_tc_combine1 = pl.pallas_call(
    _tc_combine1_body,
    out_shape=jax.ShapeDtypeStruct((NS, TW), jnp.float32),
    in_specs=[pl.BlockSpec(), pl.BlockSpec(), pl.BlockSpec(),
              pl.BlockSpec(), pl.BlockSpec(), _smem_spec, pl.BlockSpec()],
)

_tc_final = pl.pallas_call(
    _tc_final_body,
    out_shape=jax.ShapeDtypeStruct((NS, NCLS), jnp.float32),
)


def kernel(embed_X, after_nodes, edge_src1, edge_dst1, edge_rel1,
           edge_src2, edge_dst2, edge_rel2,
           basis1, comp1, root1, basis2, comp2, root2):
    i32 = jnp.int32
    an2 = after_nodes.astype(i32).reshape(NS // 128, 128)
    s1 = edge_src1.astype(i32).reshape(E // 128, 128)
    d1 = edge_dst1.astype(i32).reshape(E // 128, 128)
    r1 = edge_rel1.astype(i32).reshape(E // 128, 128)
    s2 = edge_src2.astype(i32).reshape(E // 128, 128)
    d2 = edge_dst2.astype(i32).reshape(E // 128, 128)
    r2 = edge_rel2.astype(i32).reshape(E // 128, 128)

    h0, norm1, norm2 = _sc_prep(embed_X, an2, d1, d2)
    norm1 = norm1.reshape(NS, 1)
    norm2 = norm2.reshape(NS, 1)

    table1 = _tc_table1(h0, basis1, comp1, root1)
    a0, a1 = _sc_msg(table1.reshape(NS * (N_RELS + 1), EMB), s1, r1, d1)
    table2 = _tc_combine1(a0, a1, norm1, table1, basis2, comp2, root2)
    b0, b1 = _sc_msg(table2.reshape(NS * (N_RELS + 1), EMB), s2, r2, d2)
    return _tc_final(b0, b1, norm2, table2)


# speedup vs baseline: 102.7155x; 1.6356x over previous
import functools

import jax
import jax.numpy as jnp
from jax import lax
from jax.experimental import pallas as pl
from jax.experimental.pallas import tpu as pltpu
from jax.experimental.pallas import tpu_sc as plsc

N_NODES = 10000
FEAT = 128
EMB = 16
NCLS = 16
N_RELS = 8
N_BASES = 4
NS = 8192
E = 262144

NC = 2
NSUB = 16
NW = NC * NSUB
L = 16

EPW = E // NW
ROWS_PW = EPW // 128
GRP = 8
NGRP = ROWS_PW // GRP
CHUNK = NS // NSUB

_mesh = plsc.VectorSubcoreMesh(
    core_axis_name="c", subcore_axis_name="s", num_cores=NC, num_subcores=NSUB
)
_sc_params = pltpu.CompilerParams(
    needs_layout_passes=False, use_tc_tiling_on_sc=False
)


def _zero_f32(ref, nrows):
    z = jnp.zeros((L,), jnp.float32)

    def body(i, _):
        ref[i, :] = z
        return 0

    lax.fori_loop(0, nrows, body, 0)


def _zero_flat(ref, n):
    z = jnp.zeros((L,), jnp.float32)

    def body(i, _):
        ref[pl.ds(i * L, L)] = z
        return 0

    lax.fori_loop(0, n // L, body, 0)


@functools.partial(
    pl.kernel,
    out_type=[
        jax.ShapeDtypeStruct((NS,), jnp.float32),
        jax.ShapeDtypeStruct((NS,), jnp.float32),
    ],
    mesh=_mesh,
    scratch_types=[
        pltpu.VMEM((128, 128), jnp.int32),
        pltpu.VMEM((NS,), jnp.float32),
        pltpu.VMEM((NSUB, CHUNK), jnp.float32),
        pltpu.VMEM((CHUNK,), jnp.float32),
        pltpu.VMEM_SHARED((NSUB, NS), jnp.float32),
        pltpu.SemaphoreType.DMA,
    ],
    compiler_params=_sc_params,
)
def _sc_deg(dst1_hbm, dst2_hbm, norm1_hbm, norm2_hbm,
            dst_v, deg_v, tb_v, acc_v, deg_sh, sem):
    cid = lax.axis_index("c")
    sid = lax.axis_index("s")

    _zero_flat(deg_v, NS)
    ones = jnp.full((L,), 1.0, jnp.float32)

    def count(dst_hbm):
        pltpu.sync_copy(dst_hbm.at[pl.ds(sid * 128, 128)], dst_v)

        def body(i, _):
            for k in range(8):
                d16 = dst_v[i, pl.ds(k * L, L)]
                plsc.addupdate_scatter(deg_v, [d16], ones)
            return 0

        lax.fori_loop(0, 128, body, 0)

    @pl.when(cid == 0)
    def _():
        count(dst1_hbm)

    @pl.when(cid == 1)
    def _():
        count(dst2_hbm)

    pltpu.sync_copy(deg_v, deg_sh.at[sid])
    plsc.subcore_barrier()
    for t in range(NSUB):
        pltpu.async_copy(deg_sh.at[t, pl.ds(sid * CHUNK, CHUNK)], tb_v.at[t], sem)
    for t in range(NSUB):
        pltpu.make_async_copy(
            deg_sh.at[t, pl.ds(sid * CHUNK, CHUNK)], tb_v.at[t], sem
        ).wait()

    def red(i, _):
        s = tb_v[0, pl.ds(i * L, L)]
        for t in range(1, NSUB):
            s = s + tb_v[t, pl.ds(i * L, L)]
        acc_v[pl.ds(i * L, L)] = 1.0 / jnp.maximum(s, 1.0)
        return 0

    lax.fori_loop(0, CHUNK // L, red, 0)

    @pl.when(cid == 0)
    def _():
        pltpu.sync_copy(acc_v, norm1_hbm.at[pl.ds(sid * CHUNK, CHUNK)])

    @pl.when(cid == 1)
    def _():
        pltpu.sync_copy(acc_v, norm2_hbm.at[pl.ds(sid * CHUNK, CHUNK)])


def _make_sc_msg(layer1: bool):
    scratch = [
        pltpu.VMEM((ROWS_PW, 128), jnp.int32),
        pltpu.VMEM((ROWS_PW, 128), jnp.int32),
        pltpu.VMEM((ROWS_PW, 128), jnp.int32),
        pltpu.VMEM((2, GRP * 128, EMB), jnp.float32),
        pltpu.VMEM((CHUNK, EMB), jnp.float32),
        pltpu.VMEM((CHUNK // 8, 128), jnp.float32),
        pltpu.VMEM((CHUNK,), jnp.float32),
        pltpu.VMEM_SHARED((NS, EMB), jnp.float32),
        pltpu.SemaphoreType.DMA,
        pltpu.SemaphoreType.DMA,
    ]
    if layer1:
        scratch += [
            pltpu.VMEM((NS // 128, 128), jnp.int32),
            pltpu.VMEM((CHUNK, EMB), jnp.float32),
        ]

    def body(*args):
        if layer1:
            (table_hbm, root_hbm, an_hbm, src_hbm, rel_hbm, dst_hbm, norm_hbm,
             agg0_hbm, agg1_hbm,
             src_v, rel_v, dst_v, rows, zbuf, pbuf, norm_v, agg_sh,
             semg, sems, an_v, rbuf) = args
        else:
            (table_hbm, src_hbm, rel_hbm, dst_hbm, norm_hbm,
             agg0_hbm, agg1_hbm,
             src_v, rel_v, dst_v, rows, zbuf, pbuf, norm_v, agg_sh,
             semg, sems) = args
        cid = lax.axis_index("c")
        sid = lax.axis_index("s")
        w = cid * NSUB + sid

        _zero_f32(zbuf, CHUNK)
        pltpu.sync_copy(zbuf, agg_sh.at[pl.ds(sid * CHUNK, CHUNK)])
        plsc.subcore_barrier()

        esl = pl.ds(w * ROWS_PW, ROWS_PW)
        cps = [pltpu.async_copy(src_hbm.at[esl], src_v, semg),
               pltpu.async_copy(rel_hbm.at[esl], rel_v, semg),
               pltpu.async_copy(dst_hbm.at[esl], dst_v, semg)]
        if layer1:
            cps.append(pltpu.async_copy(an_hbm, an_v, semg))
        for c in cps:
            c.wait()

        if layer1:
            def gidx(i, _):
                for k in range(128 // L):
                    sl = pl.ds(k * L, L)
                    s = src_v[i, sl]
                    a = plsc.load_gather(an_v, [s >> 7, s & 127])
                    src_v[i, sl] = (a << 3) | rel_v[i, sl]
                return 0
        else:
            def gidx(i, _):
                for k in range(128 // L):
                    sl = pl.ds(k * L, L)
                    s = src_v[i, sl]
                    t = ((s & 7) << 10) | (s >> 3)
                    src_v[i, sl] = (t << 3) | rel_v[i, sl]
                return 0

        lax.fori_loop(0, ROWS_PW, gidx, 0)

        def group(g, _):
            base = g * GRP
            pg = g & 1
            for k in range(GRP):
                pltpu.async_copy(
                    table_hbm.at[src_v.at[base + k]],
                    rows.at[pg, pl.ds(k * 128, 128)], semg,
                )

            @pl.when(g > 0)
            def _():
                for k in range(GRP):
                    pltpu.make_async_copy(
                        rows.at[1 - pg, pl.ds(k * 128, 128)],
                        agg_sh.at[dst_v.at[(g - 1) * GRP + k]], sems,
                    ).wait()

            for k in range(GRP):
                pltpu.make_async_copy(
                    table_hbm.at[src_v.at[base + k]],
                    rows.at[pg, pl.ds(k * 128, 128)], semg,
                ).wait()
            for k in range(GRP):
                pltpu.async_copy(
                    rows.at[pg, pl.ds(k * 128, 128)],
                    agg_sh.at[dst_v.at[base + k]], sems, add=True,
                )
            return 0

        lax.fori_loop(0, NGRP, group, 0)
        for k in range(GRP):
            pltpu.make_async_copy(
                rows.at[(NGRP - 1) & 1, pl.ds(k * 128, 128)],
                agg_sh.at[dst_v.at[(NGRP - 1) * GRP + k]], sems,
            ).wait()
        plsc.subcore_barrier()

        pltpu.sync_copy(norm_hbm.at[pl.ds(sid * CHUNK, CHUNK)], norm_v)
        pltpu.sync_copy(agg_sh.at[pl.ds(sid * CHUNK, CHUNK)], zbuf)

        def pack(with_root):
            def nmul(i, _):
                n16 = norm_v[pl.ds(i * L, L)]
                for j in range(L):
                    v = zbuf[i * L + j, :] * n16[j]
                    if with_root:
                        v = v + rbuf[i * L + j, :]
                    pbuf[i * 2 + j // 8, pl.ds((j % 8) * EMB, EMB)] = v
                return 0

            lax.fori_loop(0, CHUNK // L, nmul, 0)

        if layer1:
            @pl.when(cid == 0)
            def _():
                for jj in range(CHUNK // 128):
                    pltpu.async_copy(
                        root_hbm.at[an_v.at[sid * (CHUNK // 128) + jj]],
                        rbuf.at[pl.ds(jj * 128, 128)], semg,
                    )
                for jj in range(CHUNK // 128):
                    pltpu.make_async_copy(
                        root_hbm.at[an_v.at[sid * (CHUNK // 128) + jj]],
                        rbuf.at[pl.ds(jj * 128, 128)], semg,
                    ).wait()
                pack(True)

            @pl.when(cid == 1)
            def _():
                pack(False)
        else:
            pack(False)

        @pl.when(cid == 0)
        def _():
            pltpu.sync_copy(pbuf, agg0_hbm.at[pl.ds(sid * (CHUNK // 8),
                                                    CHUNK // 8)])

        @pl.when(cid == 1)
        def _():
            pltpu.sync_copy(pbuf, agg1_hbm.at[pl.ds(sid * (CHUNK // 8),
                                                    CHUNK // 8)])

    return pl.kernel(
        body,
        out_type=[
            jax.ShapeDtypeStruct((NS // 8, 128), jnp.float32),
            jax.ShapeDtypeStruct((NS // 8, 128), jnp.float32),
        ],
        mesh=_mesh,
        scratch_types=scratch,
        compiler_params=_sc_params,
    )


_sc_msg1 = _make_sc_msg(True)
_sc_msg2 = _make_sc_msg(False)


def _wmsg(basis_ref, comp_ref):
    parts = []
    for r in range(N_RELS):
        wr = comp_ref[r, 0] * basis_ref[0]
        for b in range(1, N_BASES):
            wr = wr + comp_ref[r, b] * basis_ref[b]
        parts.append(wr)
    return jnp.concatenate(parts, axis=1)


def _tc_tablef_body(x_ref, basis_ref, comp_ref, root_ref, tab_ref, rootf_ref):
    x = x_ref[...]
    wc = _wmsg(basis_ref, comp_ref)
    tab_ref[...] = jnp.dot(x, wc, preferred_element_type=jnp.float32)
    rootf_ref[...] = jnp.dot(x, root_ref[...],
                             preferred_element_type=jnp.float32)


def _tc_combine1_body(p0_ref, p1_ref, basis_ref, comp_ref, tmsg_ref, h1p_ref):
    h1p = jax.nn.relu(p0_ref[...] + p1_ref[...])
    h1p_ref[...] = h1p
    wc = _wmsg(basis_ref, comp_ref)
    tmsg_ref[...] = jnp.concatenate(
        [jnp.dot(h1p[:, j * EMB:(j + 1) * EMB], wc,
                 preferred_element_type=jnp.float32) for j in range(8)],
        axis=0)


def _tc_final_body(q0_ref, q1_ref, h1p_ref, root2_ref, out_ref):
    h1p = h1p_ref[...]
    r2p = jnp.concatenate(
        [jnp.dot(h1p[:, j * EMB:(j + 1) * EMB], root2_ref[...],
                 preferred_element_type=jnp.float32) for j in range(8)],
        axis=1)
    out_ref[...] = q0_ref[...] + q1_ref[...] + r2p


_smem_spec = pl.BlockSpec(memory_space=pltpu.SMEM)

_tc_tablef = pl.pallas_call(
    _tc_tablef_body,
    out_shape=[
        jax.ShapeDtypeStruct((N_NODES, N_RELS * EMB), jnp.float32),
        jax.ShapeDtypeStruct((N_NODES, EMB), jnp.float32),
    ],
    in_specs=[pl.BlockSpec(), pl.BlockSpec(), _smem_spec, pl.BlockSpec()],
)

_tc_combine1 = pl.pallas_call(
    _tc_combine1_body,
    out_shape=[
        jax.ShapeDtypeStruct((NS, N_RELS * EMB), jnp.float32),
        jax.ShapeDtypeStruct((NS // 8, 128), jnp.float32),
    ],
    in_specs=[pl.BlockSpec(), pl.BlockSpec(), pl.BlockSpec(), _smem_spec],
)

_tc_final = pl.pallas_call(
    _tc_final_body,
    out_shape=jax.ShapeDtypeStruct((NS // 8, 128), jnp.float32),
)


def kernel(embed_X, after_nodes, edge_src1, edge_dst1, edge_rel1,
           edge_src2, edge_dst2, edge_rel2,
           basis1, comp1, root1, basis2, comp2, root2):
    i32 = jnp.int32
    an2 = after_nodes.astype(i32).reshape(NS // 128, 128)
    s1 = edge_src1.astype(i32).reshape(E // 128, 128)
    d1 = edge_dst1.astype(i32).reshape(E // 128, 128)
    r1 = edge_rel1.astype(i32).reshape(E // 128, 128)
    s2 = edge_src2.astype(i32).reshape(E // 128, 128)
    d2 = edge_dst2.astype(i32).reshape(E // 128, 128)
    r2 = edge_rel2.astype(i32).reshape(E // 128, 128)

    tableF, rootF = _tc_tablef(embed_X, basis1, comp1, root1)
    norm1, norm2 = _sc_deg(d1, d2)

    p0, p1 = _sc_msg1(tableF.reshape(N_NODES * N_RELS, EMB), rootF,
                      an2, s1, r1, d1, norm1)
    table2, h1p = _tc_combine1(p0, p1, basis2, comp2)
    q0, q1 = _sc_msg2(table2.reshape(NS * N_RELS, EMB), s2, r2, d2, norm2)
    return _tc_final(q0, q1, h1p, root2).reshape(NS, NCLS)


# speedup vs baseline: 105.0980x; 1.0232x over previous
import functools

import jax
import jax.numpy as jnp
from jax import lax
from jax.experimental import pallas as pl
from jax.experimental.pallas import tpu as pltpu
from jax.experimental.pallas import tpu_sc as plsc

N_NODES = 10000
FEAT = 128
EMB = 16
NCLS = 16
N_RELS = 8
N_BASES = 4
NS = 8192
E = 262144

NC = 2
NSUB = 16
NW = NC * NSUB
L = 16

EPW = E // NW
ROWS_PW = EPW // 128
GRP = 8
NGRP = ROWS_PW // GRP
CHUNK = NS // NSUB

_mesh = plsc.VectorSubcoreMesh(
    core_axis_name="c", subcore_axis_name="s", num_cores=NC, num_subcores=NSUB
)
_sc_params = pltpu.CompilerParams(
    needs_layout_passes=False, use_tc_tiling_on_sc=False
)


def _zero_f32(ref, nrows):
    z = jnp.zeros((L,), jnp.float32)

    def body(i, _):
        ref[i, :] = z
        return 0

    lax.fori_loop(0, nrows, body, 0)


def _zero_flat(ref, n):
    z = jnp.zeros((L,), jnp.float32)

    def body(i, _):
        ref[pl.ds(i * L, L)] = z
        return 0

    lax.fori_loop(0, n // L, body, 0)


@functools.partial(
    pl.kernel,
    out_type=[
        jax.ShapeDtypeStruct((NS,), jnp.float32),
        jax.ShapeDtypeStruct((NS,), jnp.float32),
    ],
    mesh=_mesh,
    scratch_types=[
        pltpu.VMEM((128, 128), jnp.int32),
        pltpu.VMEM((NS,), jnp.float32),
        pltpu.VMEM((NSUB, CHUNK), jnp.float32),
        pltpu.VMEM((CHUNK,), jnp.float32),
        pltpu.VMEM_SHARED((NSUB, NS), jnp.float32),
        pltpu.SemaphoreType.DMA,
    ],
    compiler_params=_sc_params,
)
def _sc_deg(dst1_hbm, dst2_hbm, norm1_hbm, norm2_hbm,
            dst_v, deg_v, tb_v, acc_v, deg_sh, sem):
    cid = lax.axis_index("c")
    sid = lax.axis_index("s")

    _zero_flat(deg_v, NS)
    ones = jnp.full((L,), 1.0, jnp.float32)

    def count(dst_hbm):
        pltpu.sync_copy(dst_hbm.at[pl.ds(sid * 128, 128)], dst_v)

        def body(i, _):
            for k in range(8):
                d16 = dst_v[i, pl.ds(k * L, L)]
                plsc.addupdate_scatter(deg_v, [d16], ones)
            return 0

        lax.fori_loop(0, 128, body, 0)

    @pl.when(cid == 0)
    def _():
        count(dst1_hbm)

    @pl.when(cid == 1)
    def _():
        count(dst2_hbm)

    pltpu.sync_copy(deg_v, deg_sh.at[sid])
    plsc.subcore_barrier()
    for t in range(NSUB):
        pltpu.async_copy(deg_sh.at[t, pl.ds(sid * CHUNK, CHUNK)], tb_v.at[t], sem)
    for t in range(NSUB):
        pltpu.make_async_copy(
            deg_sh.at[t, pl.ds(sid * CHUNK, CHUNK)], tb_v.at[t], sem
        ).wait()

    def red(i, _):
        s = tb_v[0, pl.ds(i * L, L)]
        for t in range(1, NSUB):
            s = s + tb_v[t, pl.ds(i * L, L)]
        acc_v[pl.ds(i * L, L)] = 1.0 / jnp.maximum(s, 1.0)
        return 0

    lax.fori_loop(0, CHUNK // L, red, 0)

    @pl.when(cid == 0)
    def _():
        pltpu.sync_copy(acc_v, norm1_hbm.at[pl.ds(sid * CHUNK, CHUNK)])

    @pl.when(cid == 1)
    def _():
        pltpu.sync_copy(acc_v, norm2_hbm.at[pl.ds(sid * CHUNK, CHUNK)])


def _make_sc_msg(layer1: bool):
    scratch = [
        pltpu.VMEM((ROWS_PW, 128), jnp.int32),
        pltpu.VMEM((ROWS_PW, 128), jnp.int32),
        pltpu.VMEM((ROWS_PW, 128), jnp.int32),
        pltpu.VMEM((2, GRP * 128, EMB), jnp.float32),
        pltpu.VMEM((CHUNK, EMB), jnp.float32),
        pltpu.VMEM((CHUNK // 8, 128), jnp.float32),
        pltpu.VMEM((CHUNK,), jnp.float32),
        pltpu.VMEM_SHARED((NS, EMB), jnp.float32),
        pltpu.SemaphoreType.DMA,
        pltpu.SemaphoreType.DMA,
        pltpu.SemaphoreType.DMA,
        pltpu.VMEM((CHUNK // 8, 128), jnp.float32),
    ]
    if layer1:
        scratch += [
            pltpu.VMEM((NS // 128, 128), jnp.int32),
            pltpu.VMEM((CHUNK, EMB), jnp.float32),
        ]

    def body(*args):
        if layer1:
            (table_hbm, root_hbm, an_hbm, src_hbm, rel_hbm, dst_hbm, norm_hbm,
             agg0_hbm, agg1_hbm,
             src_v, rel_v, dst_v, rows, zbuf, pbuf, norm_v, agg_sh,
             semg, sems, semr, rbuf, an_v, rbuf1) = args
        else:
            (table_hbm, root_hbm, src_hbm, rel_hbm, dst_hbm, norm_hbm,
             agg0_hbm, agg1_hbm,
             src_v, rel_v, dst_v, rows, zbuf, pbuf, norm_v, agg_sh,
             semg, sems, semr, rbuf) = args
        cid = lax.axis_index("c")
        sid = lax.axis_index("s")
        w = cid * NSUB + sid

        _zero_f32(zbuf, CHUNK)
        pltpu.sync_copy(zbuf, agg_sh.at[pl.ds(sid * CHUNK, CHUNK)])
        plsc.subcore_barrier()

        if not layer1:
            @pl.when(cid == 0)
            def _():
                pltpu.async_copy(
                    root_hbm.at[pl.ds(sid * (CHUNK // 8), CHUNK // 8)],
                    rbuf, semr,
                )

        esl = pl.ds(w * ROWS_PW, ROWS_PW)
        cps = [pltpu.async_copy(src_hbm.at[esl], src_v, semg),
               pltpu.async_copy(rel_hbm.at[esl], rel_v, semg),
               pltpu.async_copy(dst_hbm.at[esl], dst_v, semg)]
        if layer1:
            cps.append(pltpu.async_copy(an_hbm, an_v, semg))
        for c in cps:
            c.wait()

        if layer1:
            @pl.when(cid == 0)
            def _():
                for jj in range(CHUNK // 128):
                    pltpu.async_copy(
                        root_hbm.at[an_v.at[sid * (CHUNK // 128) + jj]],
                        rbuf1.at[pl.ds(jj * 128, 128)], semr,
                    )

        if layer1:
            def gidx(i, _):
                for k in range(128 // L):
                    sl = pl.ds(k * L, L)
                    s = src_v[i, sl]
                    a = plsc.load_gather(an_v, [s >> 7, s & 127])
                    src_v[i, sl] = (a << 3) | rel_v[i, sl]
                return 0
        else:
            def gidx(i, _):
                for k in range(128 // L):
                    sl = pl.ds(k * L, L)
                    s = src_v[i, sl]
                    t = ((s & 7) << 10) | (s >> 3)
                    src_v[i, sl] = (t << 3) | rel_v[i, sl]
                return 0

        lax.fori_loop(0, ROWS_PW, gidx, 0)

        def group(g, _):
            base = g * GRP
            pg = g & 1
            for k in range(GRP):
                pltpu.async_copy(
                    table_hbm.at[src_v.at[base + k]],
                    rows.at[pg, pl.ds(k * 128, 128)], semg,
                )

            @pl.when(g > 0)
            def _():
                for k in range(GRP):
                    pltpu.make_async_copy(
                        rows.at[1 - pg, pl.ds(k * 128, 128)],
                        agg_sh.at[dst_v.at[(g - 1) * GRP + k]], sems,
                    ).wait()

            for k in range(GRP):
                pltpu.make_async_copy(
                    table_hbm.at[src_v.at[base + k]],
                    rows.at[pg, pl.ds(k * 128, 128)], semg,
                ).wait()
            for k in range(GRP):
                pltpu.async_copy(
                    rows.at[pg, pl.ds(k * 128, 128)],
                    agg_sh.at[dst_v.at[base + k]], sems, add=True,
                )
            return 0

        lax.fori_loop(0, NGRP, group, 0)
        for k in range(GRP):
            pltpu.make_async_copy(
                rows.at[(NGRP - 1) & 1, pl.ds(k * 128, 128)],
                agg_sh.at[dst_v.at[(NGRP - 1) * GRP + k]], sems,
            ).wait()
        plsc.subcore_barrier()

        pltpu.sync_copy(norm_hbm.at[pl.ds(sid * CHUNK, CHUNK)], norm_v)
        pltpu.sync_copy(agg_sh.at[pl.ds(sid * CHUNK, CHUNK)], zbuf)

        def pack(with_root):
            def nmul(i, _):
                n16 = norm_v[pl.ds(i * L, L)]
                for j in range(L):
                    r = i * 2 + j // 8
                    csl = pl.ds((j % 8) * EMB, EMB)
                    v = zbuf[i * L + j, :] * n16[j]
                    if with_root == "natural":
                        v = v + rbuf1[i * L + j, :]
                    elif with_root == "packed":
                        v = v + rbuf[r, csl]
                    pbuf[r, csl] = v
                return 0

            lax.fori_loop(0, CHUNK // L, nmul, 0)

        if layer1:
            @pl.when(cid == 0)
            def _():
                for jj in range(CHUNK // 128):
                    pltpu.make_async_copy(
                        root_hbm.at[an_v.at[sid * (CHUNK // 128) + jj]],
                        rbuf1.at[pl.ds(jj * 128, 128)], semr,
                    ).wait()
                pack("natural")

            @pl.when(cid == 1)
            def _():
                pack(None)
        else:
            @pl.when(cid == 0)
            def _():
                pltpu.make_async_copy(
                    root_hbm.at[pl.ds(sid * (CHUNK // 8), CHUNK // 8)],
                    rbuf, semr,
                ).wait()
                pack("packed")

            @pl.when(cid == 1)
            def _():
                pack(None)

        @pl.when(cid == 0)
        def _():
            pltpu.sync_copy(pbuf, agg0_hbm.at[pl.ds(sid * (CHUNK // 8),
                                                    CHUNK // 8)])

        @pl.when(cid == 1)
        def _():
            pltpu.sync_copy(pbuf, agg1_hbm.at[pl.ds(sid * (CHUNK // 8),
                                                    CHUNK // 8)])

    return pl.kernel(
        body,
        out_type=[
            jax.ShapeDtypeStruct((NS // 8, 128), jnp.float32),
            jax.ShapeDtypeStruct((NS // 8, 128), jnp.float32),
        ],
        mesh=_mesh,
        scratch_types=scratch,
        compiler_params=_sc_params,
    )


_sc_msg1 = _make_sc_msg(True)
_sc_msg2 = _make_sc_msg(False)


def _wmsg(basis_ref, comp_ref):
    parts = []
    for r in range(N_RELS):
        wr = comp_ref[r, 0] * basis_ref[0]
        for b in range(1, N_BASES):
            wr = wr + comp_ref[r, b] * basis_ref[b]
        parts.append(wr)
    return jnp.concatenate(parts, axis=1)


def _tc_tablef_body(x_ref, basis_ref, comp_ref, root_ref, tab_ref, rootf_ref):
    x = x_ref[...]
    wc = _wmsg(basis_ref, comp_ref)
    tab_ref[...] = jnp.dot(x, wc, preferred_element_type=jnp.float32)
    rootf_ref[...] = jnp.dot(x, root_ref[...],
                             preferred_element_type=jnp.float32)


def _tc_combine1_body(p0_ref, p1_ref, basis_ref, comp_ref, root2_ref,
                      tmsg_ref, r2p_ref):
    h1p = jax.nn.relu(p0_ref[...] + p1_ref[...])
    wc = _wmsg(basis_ref, comp_ref)
    tmsg_ref[...] = jnp.concatenate(
        [jnp.dot(h1p[:, j * EMB:(j + 1) * EMB], wc,
                 preferred_element_type=jnp.float32) for j in range(8)],
        axis=0)
    r2p_ref[...] = jnp.concatenate(
        [jnp.dot(h1p[:, j * EMB:(j + 1) * EMB], root2_ref[...],
                 preferred_element_type=jnp.float32) for j in range(8)],
        axis=1)


def _tc_final_body(q0_ref, q1_ref, out_ref):
    out_ref[...] = q0_ref[...] + q1_ref[...]


_smem_spec = pl.BlockSpec(memory_space=pltpu.SMEM)

_tc_tablef = pl.pallas_call(
    _tc_tablef_body,
    out_shape=[
        jax.ShapeDtypeStruct((N_NODES, N_RELS * EMB), jnp.float32),
        jax.ShapeDtypeStruct((N_NODES, EMB), jnp.float32),
    ],
    in_specs=[pl.BlockSpec(), pl.BlockSpec(), _smem_spec, pl.BlockSpec()],
)

_tc_combine1 = pl.pallas_call(
    _tc_combine1_body,
    out_shape=[
        jax.ShapeDtypeStruct((NS, N_RELS * EMB), jnp.float32),
        jax.ShapeDtypeStruct((NS // 8, 128), jnp.float32),
    ],
    in_specs=[pl.BlockSpec(), pl.BlockSpec(), pl.BlockSpec(), _smem_spec,
              pl.BlockSpec()],
)

_tc_final = pl.pallas_call(
    _tc_final_body,
    out_shape=jax.ShapeDtypeStruct((NS // 8, 128), jnp.float32),
)


def kernel(embed_X, after_nodes, edge_src1, edge_dst1, edge_rel1,
           edge_src2, edge_dst2, edge_rel2,
           basis1, comp1, root1, basis2, comp2, root2):
    i32 = jnp.int32
    an2 = after_nodes.astype(i32).reshape(NS // 128, 128)
    s1 = edge_src1.astype(i32).reshape(E // 128, 128)
    d1 = edge_dst1.astype(i32).reshape(E // 128, 128)
    r1 = edge_rel1.astype(i32).reshape(E // 128, 128)
    s2 = edge_src2.astype(i32).reshape(E // 128, 128)
    d2 = edge_dst2.astype(i32).reshape(E // 128, 128)
    r2 = edge_rel2.astype(i32).reshape(E // 128, 128)

    tableF, rootF = _tc_tablef(embed_X, basis1, comp1, root1)
    norm1, norm2 = _sc_deg(d1, d2)

    p0, p1 = _sc_msg1(tableF.reshape(N_NODES * N_RELS, EMB), rootF,
                      an2, s1, r1, d1, norm1)
    table2, r2p = _tc_combine1(p0, p1, basis2, comp2, root2)
    q0, q1 = _sc_msg2(table2.reshape(NS * N_RELS, EMB), r2p,
                      s2, r2, d2, norm2)
    return _tc_final(q0, q1).reshape(NS, NCLS)
